# baseline (speedup 1.0000x reference)
import functools

import jax
import jax.numpy as jnp
from jax import lax
from jax.experimental import pallas as pl
from jax.experimental.pallas import tpu as pltpu
from jax.experimental.pallas import tpu_sc as plsc

D_MODEL = 1024
D_VOCAB = 100000
N_TOK = 8192

NC = 2
NS = 16
NW = NC * NS
ROWS_PER_W = D_MODEL // NW
L = 16


def _sc_gather_rows(W_E, x_flat):
    mesh = plsc.VectorSubcoreMesh(core_axis_name="c", subcore_axis_name="s")

    @functools.partial(
        pl.kernel,
        out_type=jax.ShapeDtypeStruct((D_MODEL, N_TOK), jnp.float32),
        mesh=mesh,
        compiler_params=pltpu.CompilerParams(needs_layout_passes=False),
        scratch_types=[
            pltpu.VMEM((N_TOK,), jnp.int32),
            pltpu.VMEM((D_VOCAB,), jnp.float32),
            pltpu.VMEM((N_TOK,), jnp.float32),
        ],
    )
    def k(w_hbm, x_hbm, out_hbm, idx_v, row_v, g_v):
        wid = lax.axis_index("s") * NC + lax.axis_index("c")
        pltpu.sync_copy(x_hbm, idx_v)

        def row_body(r, carry):
            d = wid * ROWS_PER_W + r
            pltpu.sync_copy(w_hbm.at[d], row_v)

            def gstep(i, c):
                base = pl.multiple_of(i * L, L)
                idx16 = idx_v[pl.ds(base, L)]
                g_v[pl.ds(base, L)] = plsc.load_gather(row_v, [idx16])
                return c

            lax.fori_loop(0, N_TOK // L, gstep, 0, unroll=8)
            pltpu.sync_copy(g_v, out_hbm.at[d])
            return carry

        lax.fori_loop(0, ROWS_PER_W, row_body, 0)

    return k(W_E, x_flat)


def _tc_transpose(outT):
    BN = 512
    BD = 512

    def body(in_ref, out_ref):
        out_ref[...] = in_ref[...].T

    return pl.pallas_call(
        body,
        grid=(N_TOK // BN, D_MODEL // BD),
        in_specs=[pl.BlockSpec((BD, BN), lambda i, j: (j, i))],
        out_specs=pl.BlockSpec((BN, BD), lambda i, j: (i, j)),
        out_shape=jax.ShapeDtypeStruct((N_TOK, D_MODEL), jnp.float32),
    )(outT)


@jax.jit
def kernel(x, W_E):
    b, s = x.shape
    x_flat = x.reshape(b * s).astype(jnp.int32)
    outT = _sc_gather_rows(W_E, x_flat)
    out = _tc_transpose(outT)
    return out.reshape(b, s, D_MODEL)


# speedup vs baseline: 1.0024x; 1.0024x over previous
import functools

import jax
import jax.numpy as jnp
from jax import lax
from jax.experimental import pallas as pl
from jax.experimental.pallas import tpu as pltpu
from jax.experimental.pallas import tpu_sc as plsc

D_MODEL = 1024
D_VOCAB = 100000
N_TOK = 8192

NC = 2
NS = 16
NW = NC * NS
ROWS_PER_W = D_MODEL // NW
L = 16


def _sc_gather_rows(W_E, x_flat):
    mesh = plsc.VectorSubcoreMesh(core_axis_name="c", subcore_axis_name="s")

    @functools.partial(
        pl.kernel,
        out_type=jax.ShapeDtypeStruct((D_MODEL, N_TOK), jnp.float32),
        mesh=mesh,
        compiler_params=pltpu.CompilerParams(
            needs_layout_passes=False, use_tc_tiling_on_sc=True
        ),
        scratch_types=[
            pltpu.VMEM((N_TOK,), jnp.int32),
            pltpu.VMEM((D_VOCAB,), jnp.float32),
            pltpu.VMEM((N_TOK,), jnp.float32),
        ],
    )
    def k(w_hbm, x_hbm, out_hbm, idx_v, row_v, g_v):
        wid = lax.axis_index("s") * NC + lax.axis_index("c")
        pltpu.sync_copy(x_hbm, idx_v)

        def row_body(r, carry):
            d = wid * ROWS_PER_W + r
            pltpu.sync_copy(w_hbm.at[d], row_v)

            def gstep(i, c):
                base = pl.multiple_of(i * L, L)
                idx16 = idx_v[pl.ds(base, L)]
                g_v[pl.ds(base, L)] = plsc.load_gather(row_v, [idx16])
                return c

            lax.fori_loop(0, N_TOK // L, gstep, 0, unroll=8)
            pltpu.sync_copy(g_v, out_hbm.at[d])
            return carry

        lax.fori_loop(0, ROWS_PER_W, row_body, 0)

    return k(W_E, x_flat)


def _tc_transpose(outT):
    BN = 512
    BD = 512

    def body(in_ref, out_ref):
        out_ref[...] = in_ref[...].T

    return pl.pallas_call(
        body,
        grid=(N_TOK // BN, D_MODEL // BD),
        in_specs=[pl.BlockSpec((BD, BN), lambda i, j: (j, i))],
        out_specs=pl.BlockSpec((BN, BD), lambda i, j: (i, j)),
        out_shape=jax.ShapeDtypeStruct((N_TOK, D_MODEL), jnp.float32),
    )(outT)


@jax.jit
def kernel(x, W_E):
    b, s = x.shape
    x_flat = x.reshape(b * s).astype(jnp.int32)
    outT = _sc_gather_rows(W_E, x_flat)
    out = _tc_transpose(outT)
    return out.reshape(b, s, D_MODEL)


# speedup vs baseline: 16.0057x; 15.9678x over previous
import functools

import jax
import jax.numpy as jnp
from jax import lax
from jax.experimental import pallas as pl
from jax.experimental.pallas import tpu as pltpu
from jax.experimental.pallas import tpu_sc as plsc

D_MODEL = 1024
D_VOCAB = 100000
N_TOK = 8192

NC = 2
NS = 16
NW = NC * NS
TOK_PER_W = N_TOK // NW
CHUNK = 32
NCHUNK = TOK_PER_W // CHUNK


def _sc_row_gather(W_T, x_flat):
    mesh = plsc.VectorSubcoreMesh(core_axis_name="c", subcore_axis_name="s")

    @functools.partial(
        pl.kernel,
        out_type=jax.ShapeDtypeStruct((N_TOK, D_MODEL), jnp.float32),
        mesh=mesh,
        compiler_params=pltpu.CompilerParams(
            needs_layout_passes=False, use_tc_tiling_on_sc=True
        ),
        scratch_types=[
            pltpu.VMEM((TOK_PER_W,), jnp.int32),
            pltpu.VMEM((CHUNK, D_MODEL), jnp.float32),
            pltpu.VMEM((CHUNK, D_MODEL), jnp.float32),
            pltpu.SemaphoreType.DMA,
            pltpu.SemaphoreType.DMA,
        ],
    )
    def k(wt_hbm, x_hbm, out_hbm, idx_v, buf0, buf1, sem0, sem1):
        wid = lax.axis_index("s") * NC + lax.axis_index("c")
        base = wid * TOK_PER_W
        pltpu.sync_copy(x_hbm.at[pl.ds(base, TOK_PER_W)], idx_v)

        bufs = (buf0, buf1)
        sems = (sem0, sem1)
        copies = [None] * NCHUNK
        copies[0] = pltpu.async_copy(
            wt_hbm.at[idx_v.at[pl.ds(0, CHUNK)]], bufs[0], sems[0]
        )
        for ch in range(NCHUNK):
            if ch + 1 < NCHUNK:
                copies[ch + 1] = pltpu.async_copy(
                    wt_hbm.at[idx_v.at[pl.ds((ch + 1) * CHUNK, CHUNK)]],
                    bufs[(ch + 1) % 2],
                    sems[(ch + 1) % 2],
                )
            copies[ch].wait()
            pltpu.sync_copy(
                bufs[ch % 2], out_hbm.at[pl.ds(base + ch * CHUNK, CHUNK)]
            )

    return k(W_T, x_flat)


@jax.jit
def kernel(x, W_E):
    b, s = x.shape
    x_flat = x.reshape(b * s).astype(jnp.int32)
    out = _sc_row_gather(W_E.T, x_flat)
    return out.reshape(b, s, D_MODEL)


# speedup vs baseline: 16.2002x; 1.0122x over previous
import functools

import jax
import jax.numpy as jnp
from jax import lax
from jax.experimental import pallas as pl
from jax.experimental.pallas import tpu as pltpu
from jax.experimental.pallas import tpu_sc as plsc

D_MODEL = 1024
D_VOCAB = 100000
BATCH = 4
SEQ = 2048
N_TOK = BATCH * SEQ

NC = 2
NS = 16
NW = NC * NS
TOK_PER_W = N_TOK // NW
CHUNK = 32
NCHUNK = TOK_PER_W // CHUNK
NBUF = 3


def _sc_row_gather(W_T, x):
    mesh = plsc.VectorSubcoreMesh(core_axis_name="c", subcore_axis_name="s")

    @functools.partial(
        pl.kernel,
        out_type=jax.ShapeDtypeStruct((N_TOK, D_MODEL), jnp.float32),
        mesh=mesh,
        compiler_params=pltpu.CompilerParams(
            needs_layout_passes=False, use_tc_tiling_on_sc=True
        ),
        scratch_types=[
            pltpu.VMEM((TOK_PER_W,), jnp.int32),
            pltpu.VMEM((NBUF, CHUNK, D_MODEL), jnp.float32),
            pltpu.SemaphoreType.DMA((NBUF,)),
            pltpu.SemaphoreType.DMA((NBUF,)),
        ],
    )
    def k(wt_hbm, x_hbm, out_hbm, idx_v, bufs, gsems, wsems):
        wid = lax.axis_index("s") * NC + lax.axis_index("c")
        base = wid * TOK_PER_W
        b = wid // (SEQ // TOK_PER_W)
        p0 = (wid % (SEQ // TOK_PER_W)) * TOK_PER_W
        pltpu.sync_copy(x_hbm.at[b, pl.ds(p0, TOK_PER_W)], idx_v)

        gather = [None] * NCHUNK
        scatter = [None] * NCHUNK

        def start_gather(ch):
            return pltpu.async_copy(
                wt_hbm.at[idx_v.at[pl.ds(ch * CHUNK, CHUNK)]],
                bufs.at[ch % NBUF],
                gsems.at[ch % NBUF],
            )

        waited = [False] * NCHUNK
        for ch in range(min(NBUF - 1, NCHUNK)):
            gather[ch] = start_gather(ch)
        for ch in range(NCHUNK):
            nxt = ch + NBUF - 1
            if nxt < NCHUNK:
                if ch >= 1:
                    scatter[ch - 1].wait()
                    waited[ch - 1] = True
                gather[nxt] = start_gather(nxt)
            gather[ch].wait()
            scatter[ch] = pltpu.async_copy(
                bufs.at[ch % NBUF],
                out_hbm.at[pl.ds(base + ch * CHUNK, CHUNK)],
                wsems.at[ch % NBUF],
            )
        for ch in range(NCHUNK):
            if not waited[ch]:
                scatter[ch].wait()

    return k(W_T, x)


@jax.jit
def kernel(x, W_E):
    out = _sc_row_gather(W_E.T, x.astype(jnp.int32))
    return out.reshape(BATCH, SEQ, D_MODEL)


# speedup vs baseline: 16.5727x; 1.0230x over previous
import functools

import jax
import jax.numpy as jnp
from jax import lax
from jax.experimental import pallas as pl
from jax.experimental.pallas import tpu as pltpu
from jax.experimental.pallas import tpu_sc as plsc

D_MODEL = 1024
D_VOCAB = 100000
BATCH = 4
SEQ = 2048
N_TOK = BATCH * SEQ

NC = 2
NS = 16
NW = NC * NS
TOK_PER_W = N_TOK // NW
CHUNK = 16
NCHUNK = TOK_PER_W // CHUNK
NBUF = 7


def _sc_row_gather(W_T, x):
    mesh = plsc.VectorSubcoreMesh(core_axis_name="c", subcore_axis_name="s")

    @functools.partial(
        pl.kernel,
        out_type=jax.ShapeDtypeStruct((N_TOK, D_MODEL), jnp.float32),
        mesh=mesh,
        compiler_params=pltpu.CompilerParams(
            needs_layout_passes=False, use_tc_tiling_on_sc=True
        ),
        scratch_types=[
            pltpu.VMEM((TOK_PER_W,), jnp.int32),
            pltpu.VMEM((NBUF, CHUNK, D_MODEL), jnp.float32),
            pltpu.SemaphoreType.DMA((NBUF,)),
            pltpu.SemaphoreType.DMA((NBUF,)),
        ],
    )
    def k(wt_hbm, x_hbm, out_hbm, idx_v, bufs, gsems, wsems):
        wid = lax.axis_index("s") * NC + lax.axis_index("c")
        base = wid * TOK_PER_W
        b = wid // (SEQ // TOK_PER_W)
        p0 = (wid % (SEQ // TOK_PER_W)) * TOK_PER_W
        pltpu.sync_copy(x_hbm.at[b, pl.ds(p0, TOK_PER_W)], idx_v)

        gather = [None] * NCHUNK
        scatter = [None] * NCHUNK

        def start_gather(ch):
            return pltpu.async_copy(
                wt_hbm.at[idx_v.at[pl.ds(ch * CHUNK, CHUNK)]],
                bufs.at[ch % NBUF],
                gsems.at[ch % NBUF],
            )

        waited = [False] * NCHUNK
        for ch in range(min(NBUF - 1, NCHUNK)):
            gather[ch] = start_gather(ch)
        for ch in range(NCHUNK):
            nxt = ch + NBUF - 1
            if nxt < NCHUNK:
                if ch >= 1:
                    scatter[ch - 1].wait()
                    waited[ch - 1] = True
                gather[nxt] = start_gather(nxt)
            gather[ch].wait()
            scatter[ch] = pltpu.async_copy(
                bufs.at[ch % NBUF],
                out_hbm.at[pl.ds(base + ch * CHUNK, CHUNK)],
                wsems.at[ch % NBUF],
            )
        for ch in range(NCHUNK):
            if not waited[ch]:
                scatter[ch].wait()

    return k(W_T, x)


@jax.jit
def kernel(x, W_E):
    out = _sc_row_gather(W_E.T, x.astype(jnp.int32))
    return out.reshape(BATCH, SEQ, D_MODEL)


# speedup vs baseline: 16.6522x; 1.0048x over previous
import functools

import jax
import jax.numpy as jnp
from jax import lax
from jax.experimental import pallas as pl
from jax.experimental.pallas import tpu as pltpu
from jax.experimental.pallas import tpu_sc as plsc

D_MODEL = 1024
D_VOCAB = 100000
BATCH = 4
SEQ = 2048
N_TOK = BATCH * SEQ

NC = 2
NS = 16
NW = NC * NS
TOK_PER_W = N_TOK // NW
CHUNK = 16
NCHUNK = TOK_PER_W // CHUNK
NBUF = 7


def _sc_row_gather(W_T, x):
    mesh = plsc.VectorSubcoreMesh(core_axis_name="c", subcore_axis_name="s")

    @functools.partial(
        pl.kernel,
        out_type=jax.ShapeDtypeStruct((N_TOK, D_MODEL), jnp.float32),
        mesh=mesh,
        compiler_params=pltpu.CompilerParams(
            needs_layout_passes=False, use_tc_tiling_on_sc=True
        ),
        scratch_types=[
            pltpu.VMEM((TOK_PER_W,), jnp.int32),
            pltpu.VMEM((NBUF, CHUNK, D_MODEL), jnp.float32),
            pltpu.SemaphoreType.DMA((NBUF,)),
            pltpu.SemaphoreType.DMA((NBUF,)),
        ],
    )
    def k(wt_hbm, x_hbm, out_hbm, idx_v, bufs, gsems, wsems):
        wid = lax.axis_index("s") * NC + lax.axis_index("c")
        base = wid * TOK_PER_W
        b = wid // (SEQ // TOK_PER_W)
        p0 = (wid % (SEQ // TOK_PER_W)) * TOK_PER_W
        pltpu.sync_copy(x_hbm.at[b, pl.ds(p0, TOK_PER_W)], idx_v)

        def start_gather(ch):
            off = pl.multiple_of(ch * CHUNK, CHUNK)
            pltpu.async_copy(
                wt_hbm.at[idx_v.at[pl.ds(off, CHUNK)]],
                bufs.at[ch % NBUF],
                gsems.at[ch % NBUF],
            )

        def wait_gather(ch):
            pltpu.make_async_copy(
                wt_hbm.at[idx_v.at[pl.ds(0, CHUNK)]],
                bufs.at[ch % NBUF],
                gsems.at[ch % NBUF],
            ).wait()

        def start_scatter(ch):
            off = pl.multiple_of(base + ch * CHUNK, CHUNK)
            pltpu.async_copy(
                bufs.at[ch % NBUF],
                out_hbm.at[pl.ds(off, CHUNK)],
                wsems.at[ch % NBUF],
            )

        def wait_scatter(ch):
            pltpu.make_async_copy(
                bufs.at[ch % NBUF],
                out_hbm.at[pl.ds(base, CHUNK)],
                wsems.at[ch % NBUF],
            ).wait()

        for ch in range(NBUF - 1):
            start_gather(ch)

        def body(ch, carry):
            nxt = ch + NBUF - 1

            @pl.when(nxt < NCHUNK)
            def _():
                @pl.when(ch >= 1)
                def _():
                    wait_scatter(ch - 1)

                start_gather(nxt)

            wait_gather(ch)
            start_scatter(ch)
            return carry

        lax.fori_loop(0, NCHUNK, body, 0)
        for ch in range(NCHUNK - NBUF, NCHUNK):
            wait_scatter(ch)

    return k(W_T, x)


@jax.jit
def kernel(x, W_E):
    out = _sc_row_gather(W_E.T, x.astype(jnp.int32))
    return out.reshape(BATCH, SEQ, D_MODEL)
